# baseline (speedup 1.0000x reference)
import functools

import jax
import jax.numpy as jnp
from jax import lax
from jax.experimental import pallas as pl
from jax.experimental.pallas import tpu as pltpu
from jax.experimental.pallas import tpu_sc as plsc

_B = 16384
_D = 128

_NH = 4
_BH = _B // _NH

_NC = 2
_NS = 16
_NW = _NC * _NS
_RPW = _BH // _NW
_CH = min(256, _RPW)


@functools.cache
def _make_gather():
    mesh = plsc.VectorSubcoreMesh(core_axis_name="c", subcore_axis_name="s")
    n_chunks = _RPW // _CH

    @functools.partial(
        pl.kernel,
        mesh=mesh,
        out_type=[
            jax.ShapeDtypeStruct((_BH, _D), jnp.float32),
            jax.ShapeDtypeStruct((_BH, _D), jnp.float32),
        ],
        scratch_types=[
            pltpu.VMEM((_RPW,), jnp.int32),
            pltpu.VMEM((_RPW,), jnp.int32),
            pltpu.VMEM((_CH, _D), jnp.float32),
            pltpu.VMEM((_CH, _D), jnp.float32),
            pltpu.VMEM((_CH, _D), jnp.float32),
            pltpu.SemaphoreType.DMA,
            pltpu.SemaphoreType.DMA,
            pltpu.SemaphoreType.DMA,
            pltpu.SemaphoreType.DMA,
        ],
    )
    def _gather2(uidx_hbm, iidx_hbm, table_hbm, out_x, out_y,
                 uix_v, iix_v, bufa, bufb, bufc, gsa, gsb, gsc, st_sem):
        wid = lax.axis_index("s") * _NC + lax.axis_index("c")
        base = wid * _RPW
        pltpu.sync_copy(uidx_hbm.at[pl.ds(base, _RPW)], uix_v)
        pltpu.sync_copy(iidx_hbm.at[pl.ds(base, _RPW)], iix_v)
        sched = []
        for c in range(n_chunks):
            sched.append((uix_v, c * _CH, out_x))
        for c in range(n_chunks):
            sched.append((iix_v, c * _CH, out_y))
        bufs = [(bufa, gsa), (bufb, gsb), (bufc, gsc)]
        nb = len(bufs)
        gathers = []
        stores = []
        for k, (idx_v, off, _out) in enumerate(sched):
            buf, sem = bufs[k % nb]
            gathers.append(
                pltpu.async_copy(table_hbm.at[idx_v.at[pl.ds(off, _CH)]], buf, sem)
            )
            if k >= nb - 1:
                pidx = k - (nb - 1)
                gathers[pidx].wait()
                pbuf, _ = bufs[pidx % nb]
                _, poff, pout = sched[pidx]
                stores.append(
                    pltpu.async_copy(pbuf, pout.at[pl.ds(base + poff, _CH)], st_sem)
                )
        for pidx in range(max(0, len(sched) - (nb - 1)), len(sched)):
            gathers[pidx].wait()
            pbuf, _ = bufs[pidx % nb]
            _, poff, pout = sched[pidx]
            stores.append(
                pltpu.async_copy(pbuf, pout.at[pl.ds(base + poff, _CH)], st_sem)
            )
        for st in stores:
            st.wait()

    return _gather2


_BS = 2048


def _mlp_body(x_ref, y_ref, w1a, w1b, b1, w2, b2, w3, b3, w4, b4, o_ref):
    h = jnp.dot(x_ref[...], w1a[...], preferred_element_type=jnp.float32)
    h = h + jnp.dot(y_ref[...], w1b[...], preferred_element_type=jnp.float32)
    h = jnp.maximum(h + b1[...], 0.0)
    h = jnp.maximum(jnp.dot(h, w2[...], preferred_element_type=jnp.float32) + b2[...], 0.0)
    h = jnp.maximum(jnp.dot(h, w3[...], preferred_element_type=jnp.float32) + b3[...], 0.0)
    z = jnp.dot(h, w4[...], preferred_element_type=jnp.float32)[:, 0] + b4[0]
    o_ref[...] = 1.0 / (1.0 + jnp.exp(-z))


def _full(shape):
    return pl.BlockSpec(shape, lambda i: tuple(0 for _ in shape))


def _mlp(xg, yg, w1a, w1b, b1, w2, b2, w3, b3, w4, b4):
    return pl.pallas_call(
        _mlp_body,
        grid=(_BH // _BS,),
        in_specs=[
            pl.BlockSpec((_BS, _D), lambda i: (i, 0)),
            pl.BlockSpec((_BS, _D), lambda i: (i, 0)),
            _full((_D, 128)),
            _full((_D, 128)),
            _full((1, 128)),
            _full((128, 64)),
            _full((1, 64)),
            _full((64, 32)),
            _full((1, 32)),
            _full((32, 1)),
            _full((1,)),
        ],
        out_specs=pl.BlockSpec((_BS,), lambda i: (i,)),
        out_shape=jax.ShapeDtypeStruct((_BH,), jnp.float32),
        compiler_params=pltpu.CompilerParams(dimension_semantics=("parallel",)),
    )(xg, yg, w1a, w1b, b1, w2, b2, w3, b3, w4, b4)


def kernel(user_input, item_input, user_emb, W1, b1, W2, b2, W3, b3, W4, b4):
    uidx = user_input.astype(jnp.int32)
    iidx = item_input.astype(jnp.int32)
    gather = _make_gather()
    w1a, w1b = W1[:_D], W1[_D:]
    b1r, b2r, b3r = b1.reshape(1, -1), b2.reshape(1, -1), b3.reshape(1, -1)
    outs = []
    for h in range(_NH):
        s = slice(h * _BH, (h + 1) * _BH)
        xg, yg = gather(uidx[s], iidx[s], user_emb)
        outs.append(_mlp(xg, yg, w1a, w1b, b1r, W2, b2r, W3, b3r, W4, b4))
    return jnp.concatenate(outs)


# speedup vs baseline: 1.1366x; 1.1366x over previous
import functools

import jax
import jax.numpy as jnp
from jax import lax
from jax.experimental import pallas as pl
from jax.experimental.pallas import tpu as pltpu
from jax.experimental.pallas import tpu_sc as plsc

_B = 16384
_D = 128

_NH = 2
_BH = _B // _NH

_NC = 2
_NS = 16
_NW = _NC * _NS
_RPW = _BH // _NW
_CH = min(256, _RPW)


@functools.cache
def _make_gather():
    mesh = plsc.VectorSubcoreMesh(core_axis_name="c", subcore_axis_name="s")
    n_chunks = _RPW // _CH

    @functools.partial(
        pl.kernel,
        mesh=mesh,
        out_type=[
            jax.ShapeDtypeStruct((_BH, _D), jnp.float32),
            jax.ShapeDtypeStruct((_BH, _D), jnp.float32),
        ],
        scratch_types=[
            pltpu.VMEM((_RPW,), jnp.int32),
            pltpu.VMEM((_RPW,), jnp.int32),
            pltpu.VMEM((_CH, _D), jnp.float32),
            pltpu.VMEM((_CH, _D), jnp.float32),
            pltpu.VMEM((_CH, _D), jnp.float32),
            pltpu.SemaphoreType.DMA,
            pltpu.SemaphoreType.DMA,
            pltpu.SemaphoreType.DMA,
            pltpu.SemaphoreType.DMA,
        ],
    )
    def _gather2(uidx_hbm, iidx_hbm, table_hbm, out_x, out_y,
                 uix_v, iix_v, bufa, bufb, bufc, gsa, gsb, gsc, st_sem):
        wid = lax.axis_index("s") * _NC + lax.axis_index("c")
        base = wid * _RPW
        pltpu.sync_copy(uidx_hbm.at[pl.ds(base, _RPW)], uix_v)
        pltpu.sync_copy(iidx_hbm.at[pl.ds(base, _RPW)], iix_v)
        sched = []
        for c in range(n_chunks):
            sched.append((uix_v, c * _CH, out_x))
        for c in range(n_chunks):
            sched.append((iix_v, c * _CH, out_y))
        bufs = [(bufa, gsa), (bufb, gsb), (bufc, gsc)]
        nb = len(bufs)
        gathers = []
        stores = []
        for k, (idx_v, off, _out) in enumerate(sched):
            buf, sem = bufs[k % nb]
            gathers.append(
                pltpu.async_copy(table_hbm.at[idx_v.at[pl.ds(off, _CH)]], buf, sem)
            )
            if k >= nb - 1:
                pidx = k - (nb - 1)
                gathers[pidx].wait()
                pbuf, _ = bufs[pidx % nb]
                _, poff, pout = sched[pidx]
                stores.append(
                    pltpu.async_copy(pbuf, pout.at[pl.ds(base + poff, _CH)], st_sem)
                )
        for pidx in range(max(0, len(sched) - (nb - 1)), len(sched)):
            gathers[pidx].wait()
            pbuf, _ = bufs[pidx % nb]
            _, poff, pout = sched[pidx]
            stores.append(
                pltpu.async_copy(pbuf, pout.at[pl.ds(base + poff, _CH)], st_sem)
            )
        for st in stores:
            st.wait()

    return _gather2


_BS = 2048


def _mlp_body(x_ref, y_ref, w1a, w1b, b1, w2, b2, w3, b3, w4, b4, o_ref):
    h = jnp.dot(x_ref[...], w1a[...], preferred_element_type=jnp.float32)
    h = h + jnp.dot(y_ref[...], w1b[...], preferred_element_type=jnp.float32)
    h = jnp.maximum(h + b1[...], 0.0)
    h = jnp.maximum(jnp.dot(h, w2[...], preferred_element_type=jnp.float32) + b2[...], 0.0)
    h = jnp.maximum(jnp.dot(h, w3[...], preferred_element_type=jnp.float32) + b3[...], 0.0)
    z = jnp.dot(h, w4[...], preferred_element_type=jnp.float32)[:, 0] + b4[0]
    o_ref[...] = 1.0 / (1.0 + jnp.exp(-z))


def _full(shape):
    return pl.BlockSpec(shape, lambda i: tuple(0 for _ in shape))


def _mlp(xg, yg, w1a, w1b, b1, w2, b2, w3, b3, w4, b4):
    return pl.pallas_call(
        _mlp_body,
        grid=(_BH // _BS,),
        in_specs=[
            pl.BlockSpec((_BS, _D), lambda i: (i, 0)),
            pl.BlockSpec((_BS, _D), lambda i: (i, 0)),
            _full((_D, 128)),
            _full((_D, 128)),
            _full((1, 128)),
            _full((128, 64)),
            _full((1, 64)),
            _full((64, 32)),
            _full((1, 32)),
            _full((32, 1)),
            _full((1,)),
        ],
        out_specs=pl.BlockSpec((_BS,), lambda i: (i,)),
        out_shape=jax.ShapeDtypeStruct((_BH,), jnp.float32),
        compiler_params=pltpu.CompilerParams(dimension_semantics=("parallel",)),
    )(xg, yg, w1a, w1b, b1, w2, b2, w3, b3, w4, b4)


def kernel(user_input, item_input, user_emb, W1, b1, W2, b2, W3, b3, W4, b4):
    uidx = user_input.astype(jnp.int32)
    iidx = item_input.astype(jnp.int32)
    gather = _make_gather()
    w1a, w1b = W1[:_D], W1[_D:]
    b1r, b2r, b3r = b1.reshape(1, -1), b2.reshape(1, -1), b3.reshape(1, -1)
    outs = []
    for h in range(_NH):
        s = slice(h * _BH, (h + 1) * _BH)
        xg, yg = gather(uidx[s], iidx[s], user_emb)
        outs.append(_mlp(xg, yg, w1a, w1b, b1r, W2, b2r, W3, b3r, W4, b4))
    return jnp.concatenate(outs)


# speedup vs baseline: 1.1527x; 1.0142x over previous
import functools

import jax
import jax.numpy as jnp
from jax import lax
from jax.experimental import pallas as pl
from jax.experimental.pallas import tpu as pltpu
from jax.experimental.pallas import tpu_sc as plsc

_B = 16384
_D = 128

_NH = 2
_BH = _B // _NH

_NC = 2
_NS = 16
_NW = _NC * _NS
_RPW = _BH // _NW
_CH = min(256, _RPW)


@functools.cache
def _make_gather():
    mesh = plsc.VectorSubcoreMesh(core_axis_name="c", subcore_axis_name="s")
    n_chunks = _RPW // _CH

    @functools.partial(
        pl.kernel,
        mesh=mesh,
        out_type=[
            jax.ShapeDtypeStruct((_BH, _D), jnp.float32),
            jax.ShapeDtypeStruct((_BH, _D), jnp.float32),
        ],
        scratch_types=[
            pltpu.VMEM((_RPW,), jnp.int32),
            pltpu.VMEM((_RPW,), jnp.int32),
            pltpu.VMEM((_CH, _D), jnp.float32),
            pltpu.VMEM((_CH, _D), jnp.float32),
            pltpu.VMEM((_CH, _D), jnp.float32),
            pltpu.SemaphoreType.DMA,
            pltpu.SemaphoreType.DMA,
            pltpu.SemaphoreType.DMA,
            pltpu.SemaphoreType.DMA,
        ],
    )
    def _gather2(uidx_hbm, iidx_hbm, table_hbm, out_x, out_y,
                 uix_v, iix_v, bufa, bufb, bufc, gsa, gsb, gsc, st_sem):
        wid = lax.axis_index("s") * _NC + lax.axis_index("c")
        base = wid * _RPW
        pltpu.sync_copy(uidx_hbm.at[pl.ds(base, _RPW)], uix_v)
        pltpu.sync_copy(iidx_hbm.at[pl.ds(base, _RPW)], iix_v)
        sched = []
        for c in range(n_chunks):
            sched.append((uix_v, c * _CH, out_x))
        for c in range(n_chunks):
            sched.append((iix_v, c * _CH, out_y))
        bufs = [(bufa, gsa), (bufb, gsb), (bufc, gsc)]
        nb = len(bufs)
        gathers = []
        stores = []
        for k, (idx_v, off, _out) in enumerate(sched):
            buf, sem = bufs[k % nb]
            gathers.append(
                pltpu.async_copy(table_hbm.at[idx_v.at[pl.ds(off, _CH)]], buf, sem)
            )
            if k >= nb - 1:
                pidx = k - (nb - 1)
                gathers[pidx].wait()
                pbuf, _ = bufs[pidx % nb]
                _, poff, pout = sched[pidx]
                stores.append(
                    pltpu.async_copy(pbuf, pout.at[pl.ds(base + poff, _CH)], st_sem)
                )
        for pidx in range(max(0, len(sched) - (nb - 1)), len(sched)):
            gathers[pidx].wait()
            pbuf, _ = bufs[pidx % nb]
            _, poff, pout = sched[pidx]
            stores.append(
                pltpu.async_copy(pbuf, pout.at[pl.ds(base + poff, _CH)], st_sem)
            )
        for st in stores:
            st.wait()

    return _gather2


_BS = 4096


def _mlp_body(x_ref, y_ref, w1a, w1b, b1, w2, b2, w3, b3, w4, b4, o_ref):
    h = jnp.dot(x_ref[...], w1a[...], preferred_element_type=jnp.float32)
    h = h + jnp.dot(y_ref[...], w1b[...], preferred_element_type=jnp.float32)
    h = jnp.maximum(h + b1[...], 0.0)
    h = jnp.maximum(jnp.dot(h, w2[...], preferred_element_type=jnp.float32) + b2[...], 0.0)
    h = jnp.maximum(jnp.dot(h, w3[...], preferred_element_type=jnp.float32) + b3[...], 0.0)
    z = jnp.dot(h, w4[...], preferred_element_type=jnp.float32)[:, 0] + b4[0]
    o_ref[...] = 1.0 / (1.0 + jnp.exp(-z))


def _full(shape):
    return pl.BlockSpec(shape, lambda i: tuple(0 for _ in shape))


def _mlp(xg, yg, w1a, w1b, b1, w2, b2, w3, b3, w4, b4):
    return pl.pallas_call(
        _mlp_body,
        grid=(_BH // _BS,),
        in_specs=[
            pl.BlockSpec((_BS, _D), lambda i: (i, 0)),
            pl.BlockSpec((_BS, _D), lambda i: (i, 0)),
            _full((_D, 128)),
            _full((_D, 128)),
            _full((1, 128)),
            _full((128, 64)),
            _full((1, 64)),
            _full((64, 32)),
            _full((1, 32)),
            _full((32, 1)),
            _full((1,)),
        ],
        out_specs=pl.BlockSpec((_BS,), lambda i: (i,)),
        out_shape=jax.ShapeDtypeStruct((_BH,), jnp.float32),
        compiler_params=pltpu.CompilerParams(dimension_semantics=("parallel",)),
    )(xg, yg, w1a, w1b, b1, w2, b2, w3, b3, w4, b4)


def kernel(user_input, item_input, user_emb, W1, b1, W2, b2, W3, b3, W4, b4):
    uidx = user_input.astype(jnp.int32)
    iidx = item_input.astype(jnp.int32)
    gather = _make_gather()
    w1a, w1b = W1[:_D], W1[_D:]
    b1r, b2r, b3r = b1.reshape(1, -1), b2.reshape(1, -1), b3.reshape(1, -1)
    outs = []
    for h in range(_NH):
        s = slice(h * _BH, (h + 1) * _BH)
        xg, yg = gather(uidx[s], iidx[s], user_emb)
        outs.append(_mlp(xg, yg, w1a, w1b, b1r, W2, b2r, W3, b3r, W4, b4))
    return jnp.concatenate(outs)


# speedup vs baseline: 1.1570x; 1.0037x over previous
import functools

import jax
import jax.numpy as jnp
from jax import lax
from jax.experimental import pallas as pl
from jax.experimental.pallas import tpu as pltpu
from jax.experimental.pallas import tpu_sc as plsc

_B = 16384
_D = 128

_NH = 2
_BH = _B // _NH

_NC = 2
_NS = 16
_NW = _NC * _NS
_RPW = _BH // _NW
_CH = min(256, _RPW)


@functools.cache
def _make_gather():
    mesh = plsc.VectorSubcoreMesh(core_axis_name="c", subcore_axis_name="s")
    n_chunks = _RPW // _CH

    @functools.partial(
        pl.kernel,
        mesh=mesh,
        out_type=[
            jax.ShapeDtypeStruct((_BH, _D), jnp.float32),
            jax.ShapeDtypeStruct((_BH, _D), jnp.float32),
        ],
        scratch_types=[
            pltpu.VMEM((_RPW,), jnp.int32),
            pltpu.VMEM((_RPW,), jnp.int32),
            pltpu.VMEM((_CH, _D), jnp.float32),
            pltpu.VMEM((_CH, _D), jnp.float32),
            pltpu.VMEM((_CH, _D), jnp.float32),
            pltpu.SemaphoreType.DMA,
            pltpu.SemaphoreType.DMA,
            pltpu.SemaphoreType.DMA,
            pltpu.SemaphoreType.DMA,
        ],
    )
    def _gather2(uidx_hbm, iidx_hbm, table_hbm, out_x, out_y,
                 uix_v, iix_v, bufa, bufb, bufc, gsa, gsb, gsc, st_sem):
        wid = lax.axis_index("s") * _NC + lax.axis_index("c")
        base = wid * _RPW
        pltpu.sync_copy(uidx_hbm.at[pl.ds(base, _RPW)], uix_v)
        pltpu.sync_copy(iidx_hbm.at[pl.ds(base, _RPW)], iix_v)
        sched = []
        for c in range(n_chunks):
            sched.append((uix_v, c * _CH, out_x))
        for c in range(n_chunks):
            sched.append((iix_v, c * _CH, out_y))
        bufs = [(bufa, gsa), (bufb, gsb), (bufc, gsc)]
        nb = len(bufs)
        gathers = []
        stores = []
        for k, (idx_v, off, _out) in enumerate(sched):
            buf, sem = bufs[k % nb]
            gathers.append(
                pltpu.async_copy(table_hbm.at[idx_v.at[pl.ds(off, _CH)]], buf, sem)
            )
            if k >= nb - 1:
                pidx = k - (nb - 1)
                gathers[pidx].wait()
                pbuf, _ = bufs[pidx % nb]
                _, poff, pout = sched[pidx]
                stores.append(
                    pltpu.async_copy(pbuf, pout.at[pl.ds(base + poff, _CH)], st_sem)
                )
        for pidx in range(max(0, len(sched) - (nb - 1)), len(sched)):
            gathers[pidx].wait()
            pbuf, _ = bufs[pidx % nb]
            _, poff, pout = sched[pidx]
            stores.append(
                pltpu.async_copy(pbuf, pout.at[pl.ds(base + poff, _CH)], st_sem)
            )
        for st in stores:
            st.wait()

    return _gather2


_BS = 4096


def _mlp_body(x_ref, y_ref, w1a, w1b, b1, w2, b2, w3, b3, w4, b4, o_ref):
    h = jnp.dot(x_ref[...], w1a[...], preferred_element_type=jnp.float32)
    h = h + jnp.dot(y_ref[...], w1b[...], preferred_element_type=jnp.float32)
    h = jnp.maximum(h + b1[...], 0.0)
    h = jnp.maximum(jnp.dot(h, w2[...], preferred_element_type=jnp.float32) + b2[...], 0.0)
    h = jnp.maximum(jnp.dot(h, w3[...], preferred_element_type=jnp.float32) + b3[...], 0.0)
    z = jnp.dot(h, w4[...], preferred_element_type=jnp.float32)[:, 0] + b4[0]
    o_ref[...] = 1.0 / (1.0 + jnp.exp(-z))


def _mlp2_body(x_ref, y_ref, w1_ref, b1, w2, b2, w3, b3, w4, b4, o_ref):
    h = jnp.dot(x_ref[...], w1_ref[0], preferred_element_type=jnp.float32)
    h = h + jnp.dot(y_ref[...], w1_ref[1], preferred_element_type=jnp.float32)
    h = jnp.maximum(h + b1[...], 0.0)
    h = jnp.maximum(jnp.dot(h, w2[...], preferred_element_type=jnp.float32) + b2[...], 0.0)
    h = jnp.maximum(jnp.dot(h, w3[...], preferred_element_type=jnp.float32) + b3[...], 0.0)
    z = jnp.dot(h, w4[...], preferred_element_type=jnp.float32)[:, 0] + b4[0]
    o_ref[...] = 1.0 / (1.0 + jnp.exp(-z))


def _full(shape):
    return pl.BlockSpec(shape, lambda i: tuple(0 for _ in shape))


def _mlp(xg, yg, w1r, b1, w2, b2, w3, b3, w4, b4):
    return pl.pallas_call(
        _mlp2_body,
        grid=(_BH // _BS,),
        in_specs=[
            pl.BlockSpec((_BS, _D), lambda i: (i, 0)),
            pl.BlockSpec((_BS, _D), lambda i: (i, 0)),
            _full((2, _D, 128)),
            _full((128,)),
            _full((128, 64)),
            _full((64,)),
            _full((64, 32)),
            _full((32,)),
            _full((32, 1)),
            _full((1,)),
        ],
        out_specs=pl.BlockSpec((_BS,), lambda i: (i,)),
        out_shape=jax.ShapeDtypeStruct((_BH,), jnp.float32),
        compiler_params=pltpu.CompilerParams(dimension_semantics=("parallel",)),
    )(xg, yg, w1r, b1, w2, b2, w3, b3, w4, b4)


def kernel(user_input, item_input, user_emb, W1, b1, W2, b2, W3, b3, W4, b4):
    uidx = user_input.astype(jnp.int32)
    iidx = item_input.astype(jnp.int32)
    gather = _make_gather()
    w1r = W1.reshape(2, _D, 128)
    outs = []
    for h in range(_NH):
        s = slice(h * _BH, (h + 1) * _BH)
        xg, yg = gather(uidx[s], iidx[s], user_emb)
        outs.append(_mlp(xg, yg, w1r, b1, W2, b2, W3, b3, W4, b4))
    return jnp.concatenate(outs)


# speedup vs baseline: 1.1766x; 1.0169x over previous
import functools

import jax
import jax.numpy as jnp
from jax import lax
from jax.experimental import pallas as pl
from jax.experimental.pallas import tpu as pltpu
from jax.experimental.pallas import tpu_sc as plsc

_B = 16384
_D = 128

_NH = 2
_BH = _B // _NH

_NC = 2
_NS = 16
_NW = _NC * _NS
_RPW = _BH // _NW
_CH = min(256, _RPW)


@functools.cache
def _make_gather():
    mesh = plsc.VectorSubcoreMesh(core_axis_name="c", subcore_axis_name="s")
    n_chunks = _RPW // _CH

    @functools.partial(
        pl.kernel,
        mesh=mesh,
        out_type=[
            jax.ShapeDtypeStruct((_BH, _D), jnp.float32),
            jax.ShapeDtypeStruct((_BH, _D), jnp.float32),
        ],
        scratch_types=[
            pltpu.VMEM((_RPW,), jnp.int32),
            pltpu.VMEM((_RPW,), jnp.int32),
            pltpu.VMEM((_CH, _D), jnp.float32),
            pltpu.VMEM((_CH, _D), jnp.float32),
            pltpu.VMEM((_CH, _D), jnp.float32),
            pltpu.SemaphoreType.DMA,
            pltpu.SemaphoreType.DMA,
            pltpu.SemaphoreType.DMA,
            pltpu.SemaphoreType.DMA,
        ],
    )
    def _gather2(uidx_hbm, iidx_hbm, table_hbm, out_x, out_y,
                 uix_v, iix_v, bufa, bufb, bufc, gsa, gsb, gsc, st_sem):
        wid = lax.axis_index("s") * _NC + lax.axis_index("c")
        base = wid * _RPW
        pltpu.sync_copy(uidx_hbm.at[pl.ds(base, _RPW)], uix_v)
        pltpu.sync_copy(iidx_hbm.at[pl.ds(base, _RPW)], iix_v)
        sched = []
        for c in range(n_chunks):
            sched.append((uix_v, c * _CH, out_x))
        for c in range(n_chunks):
            sched.append((iix_v, c * _CH, out_y))
        bufs = [(bufa, gsa), (bufb, gsb), (bufc, gsc)]
        nb = len(bufs)
        gathers = []
        stores = []
        for k, (idx_v, off, _out) in enumerate(sched):
            buf, sem = bufs[k % nb]
            gathers.append(
                pltpu.async_copy(table_hbm.at[idx_v.at[pl.ds(off, _CH)]], buf, sem)
            )
            if k >= nb - 1:
                pidx = k - (nb - 1)
                gathers[pidx].wait()
                pbuf, _ = bufs[pidx % nb]
                _, poff, pout = sched[pidx]
                stores.append(
                    pltpu.async_copy(pbuf, pout.at[pl.ds(base + poff, _CH)], st_sem)
                )
        for pidx in range(max(0, len(sched) - (nb - 1)), len(sched)):
            gathers[pidx].wait()
            pbuf, _ = bufs[pidx % nb]
            _, poff, pout = sched[pidx]
            stores.append(
                pltpu.async_copy(pbuf, pout.at[pl.ds(base + poff, _CH)], st_sem)
            )
        for st in stores:
            st.wait()

    return _gather2


_BS = 4096


def _mlp_body(x_ref, y_ref, w1a, w1b, b1, w2, b2, w3, b3, w4, b4, o_ref):
    h = jnp.dot(x_ref[...], w1a[...], preferred_element_type=jnp.float32)
    h = h + jnp.dot(y_ref[...], w1b[...], preferred_element_type=jnp.float32)
    h = jnp.maximum(h + b1[...], 0.0)
    h = jnp.maximum(jnp.dot(h, w2[...], preferred_element_type=jnp.float32) + b2[...], 0.0)
    h = jnp.maximum(jnp.dot(h, w3[...], preferred_element_type=jnp.float32) + b3[...], 0.0)
    z = jnp.dot(h, w4[...], preferred_element_type=jnp.float32)[:, 0] + b4[0]
    o_ref[...] = 1.0 / (1.0 + jnp.exp(-z))


def _mlp_block(x, y, w1_ref, b1, w2, b2, w3, b3, w4, b4):
    h = jnp.dot(x, w1_ref[0], preferred_element_type=jnp.float32)
    h = h + jnp.dot(y, w1_ref[1], preferred_element_type=jnp.float32)
    h = jnp.maximum(h + b1[...], 0.0)
    h = jnp.maximum(jnp.dot(h, w2[...], preferred_element_type=jnp.float32) + b2[...], 0.0)
    h = jnp.maximum(jnp.dot(h, w3[...], preferred_element_type=jnp.float32) + b3[...], 0.0)
    z = jnp.dot(h, w4[...], preferred_element_type=jnp.float32)[:, 0] + b4[0]
    return 1.0 / (1.0 + jnp.exp(-z))


def _mlp_body(x_ref, y_ref, w1_ref, b1, w2, b2, w3, b3, w4, b4, o_ref):
    o_ref[...] = _mlp_block(x_ref[...], y_ref[...],
                            w1_ref, b1, w2, b2, w3, b3, w4, b4)


def _mlp_body_acc(prev_ref, x_ref, y_ref, w1_ref, b1, w2, b2, w3, b3, w4, b4,
                  o_ref):
    del prev_ref
    o_ref[...] = _mlp_block(x_ref[...], y_ref[...],
                            w1_ref, b1, w2, b2, w3, b3, w4, b4)


def _full(shape):
    return pl.BlockSpec(shape, lambda i: tuple(0 for _ in shape))


_NBLK = _BH // _BS

_W_SPECS = [
    _full((2, _D, 128)),
    _full((128,)),
    _full((128, 64)),
    _full((64,)),
    _full((64, 32)),
    _full((32,)),
    _full((32, 1)),
    _full((1,)),
]


def _mlp(h, xg, yg, weights, prev=None):
    xy = pl.BlockSpec((_BS, _D), lambda i: (i, 0))
    out = pl.BlockSpec((_BS,), lambda i, h=h: (h * _NBLK + i,))
    if prev is None:
        return pl.pallas_call(
            _mlp_body,
            grid=(_NBLK,),
            in_specs=[xy, xy] + _W_SPECS,
            out_specs=out,
            out_shape=jax.ShapeDtypeStruct((_B,), jnp.float32),
            compiler_params=pltpu.CompilerParams(
                dimension_semantics=("arbitrary",)),
        )(xg, yg, *weights)
    return pl.pallas_call(
        _mlp_body_acc,
        grid=(_NBLK,),
        in_specs=[pl.BlockSpec(memory_space=pl.ANY), xy, xy] + _W_SPECS,
        out_specs=out,
        out_shape=jax.ShapeDtypeStruct((_B,), jnp.float32),
        input_output_aliases={0: 0},
        compiler_params=pltpu.CompilerParams(
            dimension_semantics=("arbitrary",)),
    )(prev, xg, yg, *weights)


def kernel(user_input, item_input, user_emb, W1, b1, W2, b2, W3, b3, W4, b4):
    uidx = user_input.astype(jnp.int32)
    iidx = item_input.astype(jnp.int32)
    gather = _make_gather()
    weights = (W1.reshape(2, _D, 128), b1, W2, b2, W3, b3, W4, b4)
    out = None
    for h in range(_NH):
        s = slice(h * _BH, (h + 1) * _BH)
        xg, yg = gather(uidx[s], iidx[s], user_emb)
        out = _mlp(h, xg, yg, weights, prev=out)
    return out


# speedup vs baseline: 1.2576x; 1.0689x over previous
import functools

import jax
import jax.numpy as jnp
from jax import lax
from jax.experimental import pallas as pl
from jax.experimental.pallas import tpu as pltpu
from jax.experimental.pallas import tpu_sc as plsc

_B = 16384
_D = 128

_NH = 2
_BH = _B // _NH

_NC = 2
_NS = 16
_NW = _NC * _NS
_RPW = _BH // _NW
_CH = min(256, _RPW)


@functools.cache
def _make_gather():
    mesh = plsc.VectorSubcoreMesh(core_axis_name="c", subcore_axis_name="s")
    n_chunks = _RPW // _CH

    @functools.partial(
        pl.kernel,
        mesh=mesh,
        out_type=[
            jax.ShapeDtypeStruct((_BH, _D), jnp.float32),
            jax.ShapeDtypeStruct((_BH, _D), jnp.float32),
        ],
        scratch_types=[
            pltpu.VMEM((_RPW,), jnp.int32),
            pltpu.VMEM((_RPW,), jnp.int32),
            pltpu.VMEM((_CH, _D), jnp.float32),
            pltpu.VMEM((_CH, _D), jnp.float32),
            pltpu.VMEM((_CH, _D), jnp.float32),
            pltpu.SemaphoreType.DMA,
            pltpu.SemaphoreType.DMA,
            pltpu.SemaphoreType.DMA,
            pltpu.SemaphoreType.DMA,
        ],
    )
    def _gather2(uidx_hbm, iidx_hbm, table_hbm, out_x, out_y,
                 uix_v, iix_v, bufa, bufb, bufc, gsa, gsb, gsc, st_sem):
        wid = lax.axis_index("s") * _NC + lax.axis_index("c")
        base = wid * _RPW
        pltpu.sync_copy(uidx_hbm.at[pl.ds(base, _RPW)], uix_v)
        pltpu.sync_copy(iidx_hbm.at[pl.ds(base, _RPW)], iix_v)
        sched = []
        for c in range(n_chunks):
            sched.append((uix_v, c * _CH, out_x))
        for c in range(n_chunks):
            sched.append((iix_v, c * _CH, out_y))
        bufs = [(bufa, gsa), (bufb, gsb), (bufc, gsc)]
        nb = len(bufs)
        gathers = []
        stores = []
        for k, (idx_v, off, _out) in enumerate(sched):
            buf, sem = bufs[k % nb]
            gathers.append(
                pltpu.async_copy(table_hbm.at[idx_v.at[pl.ds(off, _CH)]], buf, sem)
            )
            if k >= nb - 1:
                pidx = k - (nb - 1)
                gathers[pidx].wait()
                pbuf, _ = bufs[pidx % nb]
                _, poff, pout = sched[pidx]
                stores.append(
                    pltpu.async_copy(pbuf, pout.at[pl.ds(base + poff, _CH)], st_sem)
                )
        for pidx in range(max(0, len(sched) - (nb - 1)), len(sched)):
            gathers[pidx].wait()
            pbuf, _ = bufs[pidx % nb]
            _, poff, pout = sched[pidx]
            stores.append(
                pltpu.async_copy(pbuf, pout.at[pl.ds(base + poff, _CH)], st_sem)
            )
        for st in stores:
            st.wait()

    return _gather2


_BS = 4096


def _mlp_body(x_ref, y_ref, w1a, w1b, b1, w2, b2, w3, b3, w4, b4, o_ref):
    h = jnp.dot(x_ref[...], w1a[...], preferred_element_type=jnp.float32)
    h = h + jnp.dot(y_ref[...], w1b[...], preferred_element_type=jnp.float32)
    h = jnp.maximum(h + b1[...], 0.0)
    h = jnp.maximum(jnp.dot(h, w2[...], preferred_element_type=jnp.float32) + b2[...], 0.0)
    h = jnp.maximum(jnp.dot(h, w3[...], preferred_element_type=jnp.float32) + b3[...], 0.0)
    z = jnp.dot(h, w4[...], preferred_element_type=jnp.float32)[:, 0] + b4[0]
    o_ref[...] = 1.0 / (1.0 + jnp.exp(-z))


def _mlp_block(x, y, w1_ref, b1, w2, b2, w3, b3, w4, b4):
    h = jnp.dot(x, w1_ref[0], preferred_element_type=jnp.float32)
    h = h + jnp.dot(y, w1_ref[1], preferred_element_type=jnp.float32)
    h = jnp.maximum(h + b1[...], 0.0)
    h = jnp.maximum(jnp.dot(h, w2[...], preferred_element_type=jnp.float32) + b2[...], 0.0)
    h = jnp.maximum(jnp.dot(h, w3[...], preferred_element_type=jnp.float32) + b3[...], 0.0)
    z = jax.lax.dot_general(
        w4[...], h, (((1,), (1,)), ((), ())),
        preferred_element_type=jnp.float32)[0] + b4[0]
    return 1.0 / (1.0 + jnp.exp(-z))


def _mlp_body(x_ref, y_ref, w1_ref, b1, w2, b2, w3, b3, w4, b4, o_ref):
    o_ref[...] = _mlp_block(x_ref[...], y_ref[...],
                            w1_ref, b1, w2, b2, w3, b3, w4, b4)


def _mlp_body_acc(prev_ref, x_ref, y_ref, w1_ref, b1, w2, b2, w3, b3, w4, b4,
                  o_ref):
    del prev_ref
    o_ref[...] = _mlp_block(x_ref[...], y_ref[...],
                            w1_ref, b1, w2, b2, w3, b3, w4, b4)


def _full(shape):
    return pl.BlockSpec(shape, lambda i: tuple(0 for _ in shape))


_NBLK = _BH // _BS

_W_SPECS = [
    _full((2, _D, 128)),
    _full((128,)),
    _full((128, 64)),
    _full((64,)),
    _full((64, 32)),
    _full((32,)),
    _full((1, 32)),
    _full((1,)),
]


def _mlp(h, xg, yg, weights, prev=None):
    xy = pl.BlockSpec((_BS, _D), lambda i: (i, 0))
    out = pl.BlockSpec((_BS,), lambda i, h=h: (h * _NBLK + i,))
    if prev is None:
        return pl.pallas_call(
            _mlp_body,
            grid=(_NBLK,),
            in_specs=[xy, xy] + _W_SPECS,
            out_specs=out,
            out_shape=jax.ShapeDtypeStruct((_B,), jnp.float32),
            compiler_params=pltpu.CompilerParams(
                dimension_semantics=("arbitrary",)),
        )(xg, yg, *weights)
    return pl.pallas_call(
        _mlp_body_acc,
        grid=(_NBLK,),
        in_specs=[pl.BlockSpec(memory_space=pl.ANY), xy, xy] + _W_SPECS,
        out_specs=out,
        out_shape=jax.ShapeDtypeStruct((_B,), jnp.float32),
        input_output_aliases={0: 0},
        compiler_params=pltpu.CompilerParams(
            dimension_semantics=("arbitrary",)),
    )(prev, xg, yg, *weights)


def kernel(user_input, item_input, user_emb, W1, b1, W2, b2, W3, b3, W4, b4):
    uidx = user_input.astype(jnp.int32)
    iidx = item_input.astype(jnp.int32)
    gather = _make_gather()
    weights = (W1.reshape(2, _D, 128), b1, W2, b2, W3, b3, W4.reshape(1, 32), b4)
    out = None
    for h in range(_NH):
        s = slice(h * _BH, (h + 1) * _BH)
        xg, yg = gather(uidx[s], iidx[s], user_emb)
        out = _mlp(h, xg, yg, weights, prev=out)
    return out
